# baseline (speedup 1.0000x reference)
import functools

import jax
import jax.numpy as jnp
from jax import lax
from jax.experimental import pallas as pl
from jax.experimental.pallas import tpu as pltpu
from jax.experimental.pallas import tpu_sc as plsc

_info = plsc.get_sparse_core_info()
_NC, _NS = _info.num_cores, _info.num_subcores
_NW = _NC * _NS


def _pack_body(lo_ref, hi_ref, out_ref):
    out_ref[...] = jnp.concatenate(
        [lo_ref[...].T, hi_ref[...].T], axis=1)


_RBLK = 16384


def _pack_split(V):
    return _RBLK * ((V // 2 + _RBLK - 1) // _RBLK)


def _pack_table(tab_t, nrows=None):
    D, V = tab_t.shape
    rblk = _RBLK
    S = _pack_split(V)
    if nrows is None:
        nrows = S
    nblk = nrows // rblk
    hib = S // rblk
    lmax = (V + rblk - 1) // rblk - 1
    return pl.pallas_call(
        _pack_body,
        grid=(nblk,),
        in_specs=[
            pl.BlockSpec((D, rblk), lambda i: (0, i)),
            pl.BlockSpec(
                (D, rblk),
                lambda i, n=hib, m=lmax: (0, jnp.minimum(i + n, m))),
        ],
        out_specs=pl.BlockSpec((rblk, 2 * D), lambda i: (i, 0)),
        out_shape=jax.ShapeDtypeStruct((nrows, 2 * D), jnp.float32),
    )(tab_t, tab_t)


_MCH = 256


@functools.lru_cache(maxsize=None)
def _make_movie_pack(D, V):
    S = 32 * _MCH * ((V // 2 + 32 * _MCH - 1) // (32 * _MCH))
    assert V // 2 <= S <= V
    rpw = S // 32
    nch = rpw // _MCH
    off = V - S
    sh = off % 128
    offa = off - sh
    assert (V - S - sh) % 128 == 0 and (rpw % 128) == 0
    mesh = plsc.VectorSubcoreMesh(core_axis_name="c", subcore_axis_name="s")

    hiw = _MCH + 128
    ntail = V - 128 * (V // 128)

    @functools.partial(
        pl.kernel,
        mesh=mesh,
        out_type=jax.ShapeDtypeStruct((S, 2 * D), jnp.float32),
        scratch_types=[
            pltpu.VMEM((D, _MCH), jnp.float32),
            pltpu.VMEM((D, hiw), jnp.float32),
            pltpu.VMEM((D, ntail), jnp.float32),
            pltpu.VMEM((_MCH, 2 * D), jnp.float32),
        ],
        compiler_params=pltpu.CompilerParams(needs_layout_passes=False),
    )
    def mpack(tab_hbm, tail_hbm, out_hbm, lo_v, hi_v, tail_v, outc_v):
        wid = lax.axis_index("s") * _NC + lax.axis_index("c")
        row0 = wid * rpw
        is_w0 = wid == 0

        @pl.when(is_w0)
        def _():
            pltpu.sync_copy(tail_hbm, tail_v)

        def cbody(c, carry):
            lo_off = row0 + c * _MCH
            pltpu.sync_copy(tab_hbm.at[:, pl.ds(lo_off, _MCH)], lo_v)
            is_last = jnp.logical_and(wid == _NW - 1, c == nch - 1)

            @pl.when(jnp.logical_not(is_last))
            def _():
                pltpu.sync_copy(
                    tab_hbm.at[:, pl.ds(lo_off + offa, hiw)], hi_v)

            @pl.when(is_last)
            def _():
                pltpu.sync_copy(
                    tab_hbm.at[:, pl.ds(lo_off + offa, _MCH)],
                    hi_v.at[:, pl.ds(0, _MCH)])

            def dbody(d, carry2):
                dvec = jnp.full((16,), d, jnp.int32)
                for j in range(_MCH // 16):
                    rows = lax.iota(jnp.int32, 16) + (j * 16)
                    plsc.store_scatter(outc_v, [rows, dvec],
                                       lo_v[d, pl.ds(j * 16, 16)])
                    plsc.store_scatter(outc_v, [rows, dvec + D],
                                       hi_v[d, pl.ds(sh + j * 16, 16)])
                return carry2

            lax.fori_loop(0, D, dbody, 0)

            @pl.when(jnp.logical_and(is_w0, c == 0))
            def _():
                def tbody(d, carry3):
                    dvec = jnp.full((16,), d, jnp.int32)
                    for j in range(ntail // 16):
                        rows = lax.iota(jnp.int32, 16) + (j * 16)
                        plsc.store_scatter(outc_v, [rows, dvec + D],
                                           tail_v[d, pl.ds(j * 16, 16)])
                    return carry3
                lax.fori_loop(0, D, tbody, 0)

            pltpu.sync_copy(outc_v, out_hbm.at[pl.ds(lo_off, _MCH)])
            return carry

        lax.fori_loop(0, nch, cbody, 0)

    return mpack, S, off, 128 * (V // 128)


@functools.lru_cache(maxsize=None)
def _make_user_pack_part(D, V, S, R0):
    rows = S - R0
    rpw = rows // 32
    nch = rpw // _MCH
    val = 128 * (V // 128)
    ntail = V - val
    clamp0 = val - _MCH
    assert rpw % 128 == 0 and R0 % 128 == 0 and S % 128 == 0
    assert R0 + S <= val and ntail <= _MCH
    mesh = plsc.VectorSubcoreMesh(core_axis_name="c", subcore_axis_name="s")

    @functools.partial(
        pl.kernel,
        mesh=mesh,
        out_type=jax.ShapeDtypeStruct((rows, 2 * D), jnp.float32),
        scratch_types=[
            pltpu.VMEM((D, _MCH), jnp.float32),
            pltpu.VMEM((D, _MCH), jnp.float32),
            pltpu.VMEM((D, ntail), jnp.float32),
            pltpu.VMEM((_MCH, 2 * D), jnp.float32),
        ],
        compiler_params=pltpu.CompilerParams(needs_layout_passes=False),
    )
    def upack(tab_hbm, tail_hbm, out_hbm, lo_v, hi_v, tail_v, outc_v):
        wid = lax.axis_index("s") * _NC + lax.axis_index("c")
        row0 = wid * rpw
        is_last = wid == _NW - 1

        @pl.when(is_last)
        def _():
            pltpu.sync_copy(tail_hbm, tail_v)

        def cbody(c, carry):
            loc = row0 + c * _MCH
            g_lo = R0 + loc
            pltpu.sync_copy(tab_hbm.at[:, pl.ds(g_lo, _MCH)], lo_v)
            hi_off = jnp.minimum(g_lo + S, clamp0)
            pltpu.sync_copy(tab_hbm.at[:, pl.ds(hi_off, _MCH)], hi_v)

            def dbody(d, carry2):
                dvec = jnp.full((16,), d, jnp.int32)
                for j in range(_MCH // 16):
                    rr = lax.iota(jnp.int32, 16) + (j * 16)
                    plsc.store_scatter(outc_v, [rr, dvec],
                                       lo_v[d, pl.ds(j * 16, 16)])
                    plsc.store_scatter(outc_v, [rr, dvec + D],
                                       hi_v[d, pl.ds(j * 16, 16)])
                return carry2

            lax.fori_loop(0, D, dbody, 0)

            @pl.when(jnp.logical_and(is_last, c == nch - 1))
            def _():
                def tbody(d, carry3):
                    dvec = jnp.full((16,), d, jnp.int32)
                    for j in range(ntail // 16):
                        rr = lax.iota(jnp.int32, 16) + (_MCH - ntail + j * 16)
                        plsc.store_scatter(outc_v, [rr, dvec + D],
                                           tail_v[d, pl.ds(j * 16, 16)])
                    return carry3
                lax.fori_loop(0, D, tbody, 0)

            pltpu.sync_copy(outc_v, out_hbm.at[pl.ds(loc, _MCH)])
            return carry

        lax.fori_loop(0, nch, cbody, 0)

    return upack


@functools.lru_cache(maxsize=None)
def _make_gather(B, D):
    assert B % (8 * _NW) == 0 and D % _info.num_lanes == 0
    bpw = B // _NW
    CH = min(bpw, 128)
    NP = bpw // CH
    mesh = plsc.VectorSubcoreMesh(core_axis_name="c", subcore_axis_name="s")

    @functools.partial(
        pl.kernel,
        mesh=mesh,
        out_type=[
            jax.ShapeDtypeStruct((B, D), jnp.float32),
            jax.ShapeDtypeStruct((B, D), jnp.float32),
            jax.ShapeDtypeStruct((B, D), jnp.float32),
        ],
        scratch_types=[
            pltpu.VMEM((bpw,), jnp.int32),
            pltpu.VMEM((bpw,), jnp.int32),
            pltpu.VMEM((bpw,), jnp.int32),
            pltpu.VMEM((2, CH, D), jnp.float32),
            pltpu.VMEM((2, CH, D), jnp.float32),
            pltpu.VMEM((2, CH, D), jnp.float32),
            pltpu.SemaphoreType.DMA,
            pltpu.SemaphoreType.DMA,
            pltpu.SemaphoreType.DMA,
        ],
    )
    def gather(aidx_hbm, bidx_hbm, midx_hbm, atab_hbm, btab_hbm, mtab_hbm,
               aout_hbm, bout_hbm, mout_hbm,
               aidx_v, bidx_v, midx_v, arows_v, brows_v, mrows_v,
               sem_a, sem_b, sem_m):
        wid = lax.axis_index("s") * _NC + lax.axis_index("c")
        base = wid * bpw
        pltpu.sync_copy(aidx_hbm.at[pl.ds(base, bpw)], aidx_v)
        pltpu.sync_copy(bidx_hbm.at[pl.ds(base, bpw)], bidx_v)
        pltpu.sync_copy(midx_hbm.at[pl.ds(base, bpw)], midx_v)
        cps = [None, None]
        for p in range(NP + 1):
            sl = p % 2
            if p < NP:
                ca = pltpu.async_copy(
                    atab_hbm.at[aidx_v.at[pl.ds(p * CH, CH)]],
                    arows_v.at[sl], sem_a)
                cb = pltpu.async_copy(
                    btab_hbm.at[bidx_v.at[pl.ds(p * CH, CH)]],
                    brows_v.at[sl], sem_b)
                cm = pltpu.async_copy(
                    mtab_hbm.at[midx_v.at[pl.ds(p * CH, CH)]],
                    mrows_v.at[sl], sem_m)
                cps[sl] = (ca, cb, cm)
            if p > 0:
                pr = (p - 1) % 2
                for cp in cps[pr]:
                    cp.wait()
                off = base + (p - 1) * CH
                pltpu.sync_copy(arows_v.at[pr], aout_hbm.at[pl.ds(off, CH)])
                pltpu.sync_copy(brows_v.at[pr], bout_hbm.at[pl.ds(off, CH)])
                pltpu.sync_copy(mrows_v.at[pr], mout_hbm.at[pl.ds(off, CH)])

    return gather


def _mlp_body(ua_ref, ub_ref, m2_ref, pu_ref, tu_ref, pm_ref,
              w1u_ref, w1m_ref, b1_ref,
              w2t_ref, b2_ref, w3r_ref, b3_ref, out_ref):
    D = w1u_ref.shape[0]
    u2 = jnp.where(tu_ref[...] > 0.5, ub_ref[...], ua_ref[...])
    m2 = m2_ref[...]
    u = jnp.where(pu_ref[...] > 0.5, u2[:, D:], u2[:, :D])
    m = jnp.where(pm_ref[...] > 0.5, m2[:, D:], m2[:, :D])
    x = jnp.dot(u, w1u_ref[...], preferred_element_type=jnp.float32)
    x = x + jnp.dot(m, w1m_ref[...], preferred_element_type=jnp.float32)
    x = jnp.maximum(x + b1_ref[...], 0.0)
    x = jnp.maximum(
        jnp.dot(x, w2t_ref[...], preferred_element_type=jnp.float32) + b2_ref[...],
        0.0)
    out_ref[...] = jnp.sum(x * w3r_ref[...], axis=1, keepdims=True) + b3_ref[...]


def kernel(user, movie, user_table, movie_table, W1, b1, W2, b2, W3, b3):
    B = user.shape[0]
    NU, D = user_table.shape
    NM = movie_table.shape[0]
    H1 = W1.shape[0]
    H2 = W2.shape[0]
    NU2 = _pack_split(NU)

    user = user.astype(jnp.int32)
    movie = movie.astype(jnp.int32)

    RTC = 20 * _RBLK
    ut_t = user_table.T
    uval = 128 * (NU // 128)
    ua = _pack_table(ut_t, RTC)
    upack = _make_user_pack_part(D, NU, NU2, RTC)
    ub = upack(ut_t, ut_t[:, uval:])
    mpack, NM2, moff, mtail0 = _make_movie_pack(D, NM)
    mt_t = movie_table.T
    mt2 = mpack(mt_t, mt_t[:, mtail0:])

    pu_b = user >= NU2
    grow = jnp.where(user >= uval, NU2 - (NU - uval) + (user - uval),
                     jnp.where(pu_b, user - NU2, user))
    tu_b = grow >= RTC
    arow = jnp.clip(grow, 0, RTC - 1)
    brow = jnp.clip(grow - RTC, 0, NU2 - RTC - 1)
    mrow = jnp.where(movie >= mtail0, movie - mtail0,
                     jnp.where(movie >= NM2, movie - moff, movie))
    pu = pu_b.astype(jnp.float32).reshape(B, 1)
    tu = tu_b.astype(jnp.float32).reshape(B, 1)
    pm = (movie >= NM2).astype(jnp.float32).reshape(B, 1)

    ua_rows, ub_rows, m_rows = _make_gather(B, 2 * D)(
        arow, brow, mrow, ua, ub, mt2)

    w1u = W1[:, :D].T
    w1m = W1[:, D:].T
    w2t = W2.T
    w3r = W3.reshape(1, H2)
    b1r = b1.reshape(1, H1)
    b2r = b2.reshape(1, H2)
    b3r = b3.reshape(1, 1)

    BLK = 2048
    out = pl.pallas_call(
        _mlp_body,
        grid=(B // BLK,),
        in_specs=[
            pl.BlockSpec((BLK, 2 * D), lambda i: (i, 0)),
            pl.BlockSpec((BLK, 2 * D), lambda i: (i, 0)),
            pl.BlockSpec((BLK, 2 * D), lambda i: (i, 0)),
            pl.BlockSpec((BLK, 1), lambda i: (i, 0)),
            pl.BlockSpec((BLK, 1), lambda i: (i, 0)),
            pl.BlockSpec((BLK, 1), lambda i: (i, 0)),
            pl.BlockSpec((D, H1), lambda i: (0, 0)),
            pl.BlockSpec((D, H1), lambda i: (0, 0)),
            pl.BlockSpec((1, H1), lambda i: (0, 0)),
            pl.BlockSpec((H1, H2), lambda i: (0, 0)),
            pl.BlockSpec((1, H2), lambda i: (0, 0)),
            pl.BlockSpec((1, H2), lambda i: (0, 0)),
            pl.BlockSpec((1, 1), lambda i: (0, 0)),
        ],
        out_specs=pl.BlockSpec((BLK, 1), lambda i: (i, 0)),
        out_shape=jax.ShapeDtypeStruct((B, 1), jnp.float32),
    )(ua_rows, ub_rows, m_rows, pu, tu, pm, w1u, w1m, b1r, w2t, b2r,
      w3r, b3r)
    return out


# speedup vs baseline: 4.3110x; 4.3110x over previous
import functools

import jax
import jax.numpy as jnp
from jax import lax
from jax.experimental import pallas as pl
from jax.experimental.pallas import tpu as pltpu
from jax.experimental.pallas import tpu_sc as plsc

_info = plsc.get_sparse_core_info()
_NC, _NS = _info.num_cores, _info.num_subcores
_NW = _NC * _NS


def _pack_body(lo_ref, hi_ref, out_ref):
    out_ref[...] = jnp.concatenate(
        [lo_ref[...].T, hi_ref[...].T], axis=1)


_RBLK = 16384


def _pack_split(V):
    return _RBLK * ((V // 2 + _RBLK - 1) // _RBLK)


def _pack_table(tab_t):
    D, V = tab_t.shape
    rblk = _RBLK
    S = _pack_split(V)
    nblk = S // rblk
    lmax = (V + rblk - 1) // rblk - 1
    return pl.pallas_call(
        _pack_body,
        grid=(nblk,),
        in_specs=[
            pl.BlockSpec((D, rblk), lambda i: (0, i)),
            pl.BlockSpec(
                (D, rblk),
                lambda i, n=nblk, m=lmax: (0, jnp.minimum(i + n, m))),
        ],
        out_specs=pl.BlockSpec((rblk, 2 * D), lambda i: (i, 0)),
        out_shape=jax.ShapeDtypeStruct((S, 2 * D), jnp.float32),
    )(tab_t, tab_t)


_MCH = 256


@functools.lru_cache(maxsize=None)
def _make_movie_pack(D, V):
    S = 32 * _MCH * ((V // 2 + 32 * _MCH - 1) // (32 * _MCH))
    assert V // 2 <= S <= V
    rpw = S // 32
    nch = rpw // _MCH
    off = V - S
    sh = off % 128
    offa = off - sh
    assert (V - S - sh) % 128 == 0 and (rpw % 128) == 0
    mesh = plsc.VectorSubcoreMesh(core_axis_name="c", subcore_axis_name="s")

    hiw = _MCH + 128
    ntail = V - 128 * (V // 128)

    @functools.partial(
        pl.kernel,
        mesh=mesh,
        out_type=jax.ShapeDtypeStruct((S, 2 * D), jnp.float32),
        scratch_types=[
            pltpu.VMEM((D, _MCH), jnp.float32),
            pltpu.VMEM((D, hiw), jnp.float32),
            pltpu.VMEM((D, ntail), jnp.float32),
            pltpu.VMEM((_MCH, 2 * D), jnp.float32),
        ],
        compiler_params=pltpu.CompilerParams(needs_layout_passes=False),
    )
    def mpack(tab_hbm, tail_hbm, out_hbm, lo_v, hi_v, tail_v, outc_v):
        wid = lax.axis_index("s") * _NC + lax.axis_index("c")
        row0 = wid * rpw
        is_w0 = wid == 0

        @pl.when(is_w0)
        def _():
            pltpu.sync_copy(tail_hbm, tail_v)

        def cbody(c, carry):
            lo_off = row0 + c * _MCH
            pltpu.sync_copy(tab_hbm.at[:, pl.ds(lo_off, _MCH)], lo_v)
            is_last = jnp.logical_and(wid == _NW - 1, c == nch - 1)

            @pl.when(jnp.logical_not(is_last))
            def _():
                pltpu.sync_copy(
                    tab_hbm.at[:, pl.ds(lo_off + offa, hiw)], hi_v)

            @pl.when(is_last)
            def _():
                pltpu.sync_copy(
                    tab_hbm.at[:, pl.ds(lo_off + offa, _MCH)],
                    hi_v.at[:, pl.ds(0, _MCH)])

            def dbody(d, carry2):
                dvec = jnp.full((16,), d, jnp.int32)
                for j in range(_MCH // 16):
                    rows = lax.iota(jnp.int32, 16) + (j * 16)
                    plsc.store_scatter(outc_v, [rows, dvec],
                                       lo_v[d, pl.ds(j * 16, 16)])
                    plsc.store_scatter(outc_v, [rows, dvec + D],
                                       hi_v[d, pl.ds(sh + j * 16, 16)])
                return carry2

            lax.fori_loop(0, D, dbody, 0)

            @pl.when(jnp.logical_and(is_w0, c == 0))
            def _():
                def tbody(d, carry3):
                    dvec = jnp.full((16,), d, jnp.int32)
                    for j in range(ntail // 16):
                        rows = lax.iota(jnp.int32, 16) + (j * 16)
                        plsc.store_scatter(outc_v, [rows, dvec + D],
                                           tail_v[d, pl.ds(j * 16, 16)])
                    return carry3
                lax.fori_loop(0, D, tbody, 0)

            pltpu.sync_copy(outc_v, out_hbm.at[pl.ds(lo_off, _MCH)])
            return carry

        lax.fori_loop(0, nch, cbody, 0)

    return mpack, S, off, 128 * (V // 128)


@functools.lru_cache(maxsize=None)
def _make_gather(B, D):
    assert B % (8 * _NW) == 0 and D % _info.num_lanes == 0
    bpw = B // _NW
    CH = min(bpw, 128)
    NP = bpw // CH
    mesh = plsc.VectorSubcoreMesh(core_axis_name="c", subcore_axis_name="s")

    @functools.partial(
        pl.kernel,
        mesh=mesh,
        out_type=[
            jax.ShapeDtypeStruct((B, D), jnp.float32),
            jax.ShapeDtypeStruct((B, D), jnp.float32),
        ],
        scratch_types=[
            pltpu.VMEM((bpw,), jnp.int32),
            pltpu.VMEM((bpw,), jnp.int32),
            pltpu.VMEM((2, CH, D), jnp.float32),
            pltpu.VMEM((2, CH, D), jnp.float32),
            pltpu.SemaphoreType.DMA,
            pltpu.SemaphoreType.DMA,
        ],
    )
    def gather(uidx_hbm, midx_hbm, utab_hbm, mtab_hbm, uout_hbm, mout_hbm,
               uidx_v, midx_v, urows_v, mrows_v, sem_u, sem_m):
        wid = lax.axis_index("s") * _NC + lax.axis_index("c")
        base = wid * bpw
        pltpu.sync_copy(uidx_hbm.at[pl.ds(base, bpw)], uidx_v)
        pltpu.sync_copy(midx_hbm.at[pl.ds(base, bpw)], midx_v)
        cps = [None, None]
        for p in range(NP + 1):
            sl = p % 2
            if p < NP:
                cu = pltpu.async_copy(
                    utab_hbm.at[uidx_v.at[pl.ds(p * CH, CH)]],
                    urows_v.at[sl], sem_u)
                cm = pltpu.async_copy(
                    mtab_hbm.at[midx_v.at[pl.ds(p * CH, CH)]],
                    mrows_v.at[sl], sem_m)
                cps[sl] = (cu, cm)
            if p > 0:
                pr = (p - 1) % 2
                cu_p, cm_p = cps[pr]
                cu_p.wait()
                cm_p.wait()
                off = base + (p - 1) * CH
                pltpu.sync_copy(urows_v.at[pr], uout_hbm.at[pl.ds(off, CH)])
                pltpu.sync_copy(mrows_v.at[pr], mout_hbm.at[pl.ds(off, CH)])

    return gather


def _mlp_body(u2_ref, m2_ref, pu_ref, pm_ref, w1u_ref, w1m_ref, b1_ref,
              w2t_ref, b2_ref, w3r_ref, b3_ref, out_ref):
    D = w1u_ref.shape[0]
    u2 = u2_ref[...]
    m2 = m2_ref[...]
    u = jnp.where(pu_ref[...] > 0.5, u2[:, D:], u2[:, :D])
    m = jnp.where(pm_ref[...] > 0.5, m2[:, D:], m2[:, :D])
    x = jnp.dot(u, w1u_ref[...], preferred_element_type=jnp.float32)
    x = x + jnp.dot(m, w1m_ref[...], preferred_element_type=jnp.float32)
    x = jnp.maximum(x + b1_ref[...], 0.0)
    x = jnp.maximum(
        jnp.dot(x, w2t_ref[...], preferred_element_type=jnp.float32) + b2_ref[...],
        0.0)
    out_ref[...] = jnp.sum(x * w3r_ref[...], axis=1, keepdims=True) + b3_ref[...]


def kernel(user, movie, user_table, movie_table, W1, b1, W2, b2, W3, b3):
    B = user.shape[0]
    NU, D = user_table.shape
    NM = movie_table.shape[0]
    H1 = W1.shape[0]
    H2 = W2.shape[0]
    NU2 = _pack_split(NU)

    user = user.astype(jnp.int32)
    movie = movie.astype(jnp.int32)

    ut2 = _pack_table(user_table.T)
    mpack, NM2, moff, mtail0 = _make_movie_pack(D, NM)
    mt_t = movie_table.T
    mt2 = mpack(mt_t, mt_t[:, mtail0:])

    urow = jnp.where(user >= NU2, user - NU2, user)
    mrow = jnp.where(movie >= mtail0, movie - mtail0,
                     jnp.where(movie >= NM2, movie - moff, movie))
    pu = (user >= NU2).astype(jnp.float32).reshape(B, 1)
    pm = (movie >= NM2).astype(jnp.float32).reshape(B, 1)

    u_rows, m_rows = _make_gather(B, 2 * D)(urow, mrow, ut2, mt2)

    w1u = W1[:, :D].T
    w1m = W1[:, D:].T
    w2t = W2.T
    w3r = W3.reshape(1, H2)
    b1r = b1.reshape(1, H1)
    b2r = b2.reshape(1, H2)
    b3r = b3.reshape(1, 1)

    BLK = 2048
    out = pl.pallas_call(
        _mlp_body,
        grid=(B // BLK,),
        in_specs=[
            pl.BlockSpec((BLK, 2 * D), lambda i: (i, 0)),
            pl.BlockSpec((BLK, 2 * D), lambda i: (i, 0)),
            pl.BlockSpec((BLK, 1), lambda i: (i, 0)),
            pl.BlockSpec((BLK, 1), lambda i: (i, 0)),
            pl.BlockSpec((D, H1), lambda i: (0, 0)),
            pl.BlockSpec((D, H1), lambda i: (0, 0)),
            pl.BlockSpec((1, H1), lambda i: (0, 0)),
            pl.BlockSpec((H1, H2), lambda i: (0, 0)),
            pl.BlockSpec((1, H2), lambda i: (0, 0)),
            pl.BlockSpec((1, H2), lambda i: (0, 0)),
            pl.BlockSpec((1, 1), lambda i: (0, 0)),
        ],
        out_specs=pl.BlockSpec((BLK, 1), lambda i: (i, 0)),
        out_shape=jax.ShapeDtypeStruct((B, 1), jnp.float32),
    )(u_rows, m_rows, pu, pm, w1u, w1m, b1r, w2t, b2r, w3r, b3r)
    return out
